# baseline (speedup 1.0000x reference)
import functools

import jax
import jax.numpy as jnp
from jax import lax
from jax.experimental import pallas as pl
from jax.experimental.pallas import tpu as pltpu
from jax.experimental.pallas import tpu_sc as plsc

NUM_TYPES = 100
BATCH = 1024
DUP = 2
NUM_HEADS = 16
LENGTH = 20
HEAD_DIM = 64

NC = 2
NS = 16
NW = NC * NS
B_PER_W = BATCH // NW

K = B_PER_W * DUP
Q = 2
L = 1


def _gather_body(table_hbm, idx_hbm, out_hbm, idx_v, rows_v, gsem0,
                 gsem1, psem0, psem1):
    gsem = (gsem0, gsem1)
    psem = (psem0, psem1)
    wid = lax.axis_index("s") * NC + lax.axis_index("c")
    base = wid * K

    pltpu.sync_copy(idx_hbm.at[pl.ds(base * 8, K * 8 + 16)], idx_v)

    def gather_cp(k, b):
        half = idx_v[pl.ds(8 * k, 16)][0]
        return pltpu.make_async_copy(
            table_hbm.at[pl.ds(half, 1)], rows_v.at[b], gsem[b]
        )

    def put_cp(k, b):
        return pltpu.make_async_copy(
            rows_v.at[b], out_hbm.at[pl.ds(base + k, 1)], psem[b]
        )

    for k0 in range(L):
        gather_cp(k0, k0).start()

    for k in range(Q - L):
        gather_cp(k + L, (k + L) % Q).start()
        gather_cp(k, k % Q).wait()
        put_cp(k, k % Q).start()

    def step(i, carry):
        for t in range(Q):
            k = Q * i + (Q - L) + t
            bg = t
            bk = (t + Q - L) % Q
            put_cp(k - (Q - L), bg).wait()
            gather_cp(k + L, bg).start()
            gather_cp(k, bk).wait()
            put_cp(k, bk).start()
        return carry

    lax.fori_loop(0, (K - Q) // Q, step, 0)

    for k in range(K - L, K):
        put_cp(k - (Q - L), (k + L) % Q).wait()
        gather_cp(k, k % Q).wait()
        put_cp(k, k % Q).start()

    for k in range(K - (Q - L), K):
        put_cp(k, k % Q).wait()


def _gather(table, idx):
    mesh = plsc.VectorSubcoreMesh(core_axis_name="c", subcore_axis_name="s")
    return pl.kernel(
        _gather_body,
        out_type=jax.ShapeDtypeStruct(
            (BATCH * DUP, NUM_HEADS, LENGTH, HEAD_DIM), jnp.float32),
        mesh=mesh,
        scratch_types=[
            pltpu.VMEM((K * 8 + 16,), jnp.int32),
            pltpu.VMEM((Q, 1, NUM_HEADS, LENGTH, HEAD_DIM), jnp.float32),
            pltpu.SemaphoreType.DMA,
            pltpu.SemaphoreType.DMA,
            pltpu.SemaphoreType.DMA,
            pltpu.SemaphoreType.DMA,
        ],
    )(table, idx)


def kernel(customer_type_batch, prompt_table):
    idx = customer_type_batch.astype(jnp.int32)
    eidx = jnp.repeat(
        (idx[:, None] * DUP
         + jnp.arange(DUP, dtype=jnp.int32)[None, :]).reshape(-1), 8)
    eidx = jnp.concatenate([eidx, jnp.zeros((16,), jnp.int32)])
    table = prompt_table.reshape(NUM_TYPES * DUP, NUM_HEADS, LENGTH, HEAD_DIM)
    out = _gather(table, eidx)
    return out.reshape(BATCH, DUP, 1, NUM_HEADS, LENGTH, HEAD_DIM)


# speedup vs baseline: 1.6119x; 1.6119x over previous
import functools

import jax
import jax.numpy as jnp
from jax import lax
from jax.experimental import pallas as pl
from jax.experimental.pallas import tpu as pltpu
from jax.experimental.pallas import tpu_sc as plsc

NUM_TYPES = 100
BATCH = 1024
DUP = 2
NUM_HEADS = 16
LENGTH = 20
HEAD_DIM = 64
ROW = DUP * 1 * NUM_HEADS * LENGTH * HEAD_DIM

NC = 2
NS = 16
NW = NC * NS
B_PER_W = BATCH // NW

P = 1
PIECE = ROW // P
K = B_PER_W * P
Q = 2
L = 1


def _gather_body(table_hbm, eidx_hbm, out_hbm, eidx_v, rows_v, *sems):
    gsem = sems[:Q]
    psem = sems[Q:]
    wid = lax.axis_index("s") * NC + lax.axis_index("c")
    base = wid * K

    pltpu.sync_copy(eidx_hbm.at[pl.ds(base * 8, K * 8 + 16)], eidx_v)

    def gather_cp(k, b):
        row = eidx_v[pl.ds(8 * k, 16)][0]
        return pltpu.make_async_copy(
            table_hbm.at[pl.ds(row, 1)], rows_v.at[b], gsem[b]
        )

    def put_cp(k, b):
        return pltpu.make_async_copy(
            rows_v.at[b], out_hbm.at[pl.ds(base + k, 1)], psem[b]
        )

    for k0 in range(L):
        gather_cp(k0, k0).start()

    for k in range(Q - L):
        gather_cp(k + L, (k + L) % Q).start()
        gather_cp(k, k % Q).wait()
        put_cp(k, k % Q).start()

    def step(i, carry):
        for t in range(Q):
            k = Q * i + (Q - L) + t
            bg = t
            bk = (t + Q - L) % Q
            put_cp(k - (Q - L), bg).wait()
            gather_cp(k + L, bg).start()
            gather_cp(k, bk).wait()
            put_cp(k, bk).start()
        return carry

    lax.fori_loop(0, (K - Q) // Q, step, 0)

    for k in range(K - L, K):
        put_cp(k - (Q - L), (k + L) % Q).wait()
        gather_cp(k, k % Q).wait()
        put_cp(k, k % Q).start()

    for k in range(K - (Q - L), K):
        put_cp(k, k % Q).wait()


@functools.partial(jax.jit, static_argnames=())
def _gather(table, eidx):
    mesh = plsc.VectorSubcoreMesh(core_axis_name="c", subcore_axis_name="s")
    return pl.kernel(
        _gather_body,
        out_type=jax.ShapeDtypeStruct((BATCH * P, PIECE), jnp.float32),
        mesh=mesh,
        scratch_types=[
            pltpu.VMEM((K * 8 + 16,), jnp.int32),
            pltpu.VMEM((Q, 1, PIECE), jnp.float32),
        ] + [pltpu.SemaphoreType.DMA] * (2 * Q),
    )(table, eidx)


def kernel(customer_type_batch, prompt_table):
    idx = customer_type_batch.astype(jnp.int32)
    eidx = jnp.repeat(idx[:, None] * P + jnp.arange(P, dtype=jnp.int32)[None, :], 8)
    eidx = jnp.concatenate([eidx, jnp.zeros((16,), jnp.int32)])
    table = prompt_table.reshape(NUM_TYPES * P, PIECE)
    out = _gather(table, eidx)
    return out.reshape(BATCH, DUP, 1, NUM_HEADS, LENGTH, HEAD_DIM)
